# Initial kernel scaffold; baseline (speedup 1.0000x reference)
import jax
import jax.numpy as jnp
from jax.experimental import pallas as pl


def kernel(gene_indices, table):
    raise NotImplementedError("write your pallas kernel here")


# speedup vs baseline: 1.4680x; 1.4680x over previous
import jax
import jax.numpy as jnp
from jax.experimental import pallas as pl
from jax.experimental.pallas import tpu as pltpu
from jax.experimental.pallas import tpu_sc as plsc

_WINDOW = 512


def _sc_gather(table, flat_idx):
    num_indices = flat_idx.shape[1]
    d = table.shape[1]
    mesh = plsc.VectorSubcoreMesh(core_axis_name="c", subcore_axis_name="s")

    @pl.kernel(
        out_type=jax.ShapeDtypeStruct((num_indices, d), table.dtype),
        mesh=mesh,
        compiler_params=pltpu.CompilerParams(use_tc_tiling_on_sc=False),
    )
    def gather_kernel(table_hbm, idx_hbm, out_hbm):
        def body(idx_vmem, out_vmem):
            pltpu.sync_copy(table_hbm.at[idx_vmem.at[0]], out_vmem)

        pltpu.emit_pipeline(
            body,
            grid=(num_indices // _WINDOW,),
            in_specs=[pl.BlockSpec((1, _WINDOW), index_map=lambda i: (0, i))],
            out_specs=[pl.BlockSpec((_WINDOW, d), index_map=lambda i: (i, 0))],
            core_axis_name=("c", "s"),
            dimension_semantics=(pltpu.PARALLEL,),
        )(idx_hbm, out_hbm)

    return gather_kernel(table, flat_idx)


@jax.jit
def kernel(gene_indices, table):
    b, s = gene_indices.shape
    flat_idx = gene_indices.reshape(1, b * s).astype(jnp.int32)
    out = _sc_gather(table, flat_idx)
    return out.reshape(b, s, table.shape[1])


# speedup vs baseline: 1.4923x; 1.0166x over previous
import jax
import jax.numpy as jnp
from jax.experimental import pallas as pl
from jax.experimental.pallas import tpu as pltpu
from jax.experimental.pallas import tpu_sc as plsc

_WINDOW = 1024


def _sc_gather(table, flat_idx):
    num_indices = flat_idx.shape[1]
    d = table.shape[1]
    mesh = plsc.VectorSubcoreMesh(core_axis_name="c", subcore_axis_name="s")

    @pl.kernel(
        out_type=jax.ShapeDtypeStruct((num_indices, d), table.dtype),
        mesh=mesh,
        compiler_params=pltpu.CompilerParams(use_tc_tiling_on_sc=False),
    )
    def gather_kernel(table_hbm, idx_hbm, out_hbm):
        def body(idx_vmem, out_vmem):
            pltpu.sync_copy(table_hbm.at[idx_vmem.at[0]], out_vmem)

        pltpu.emit_pipeline(
            body,
            grid=(num_indices // _WINDOW,),
            in_specs=[pl.BlockSpec((1, _WINDOW), index_map=lambda i: (0, i))],
            out_specs=[pl.BlockSpec((_WINDOW, d), index_map=lambda i: (i, 0))],
            core_axis_name=("c", "s"),
            dimension_semantics=(pltpu.PARALLEL,),
        )(idx_hbm, out_hbm)

    return gather_kernel(table, flat_idx)


@jax.jit
def kernel(gene_indices, table):
    b, s = gene_indices.shape
    flat_idx = gene_indices.reshape(1, b * s).astype(jnp.int32)
    out = _sc_gather(table, flat_idx)
    return out.reshape(b, s, table.shape[1])


# speedup vs baseline: 1.4973x; 1.0033x over previous
import jax
import jax.numpy as jnp
from jax.experimental import pallas as pl
from jax.experimental.pallas import tpu as pltpu
from jax.experimental.pallas import tpu_sc as plsc

_WINDOW = 1024


def _sc_gather(table, flat_idx):
    num_indices = flat_idx.shape[1]
    d = table.shape[1]
    mesh = plsc.VectorSubcoreMesh(core_axis_name="c", subcore_axis_name="s")

    @pl.kernel(
        out_type=jax.ShapeDtypeStruct((num_indices, d), table.dtype),
        mesh=mesh,
        scratch_types=[pltpu.SemaphoreType.DMA, pltpu.SemaphoreType.DMA],
        compiler_params=pltpu.CompilerParams(use_tc_tiling_on_sc=False),
    )
    def gather_kernel(table_hbm, idx_hbm, out_hbm, sem0, sem1):
        half = _WINDOW // 2

        def body(idx_vmem, out_vmem):
            c0 = pltpu.make_async_copy(
                table_hbm.at[idx_vmem.at[0, pl.ds(0, half)]],
                out_vmem.at[pl.ds(0, half)],
                sem0,
            )
            c1 = pltpu.make_async_copy(
                table_hbm.at[idx_vmem.at[0, pl.ds(half, half)]],
                out_vmem.at[pl.ds(half, half)],
                sem1,
            )
            c0.start()
            c1.start()
            c0.wait()
            c1.wait()

        pltpu.emit_pipeline(
            body,
            grid=(num_indices // _WINDOW,),
            in_specs=[pl.BlockSpec((1, _WINDOW), index_map=lambda i: (0, i))],
            out_specs=[pl.BlockSpec((_WINDOW, d), index_map=lambda i: (i, 0))],
            core_axis_name=("c", "s"),
            dimension_semantics=(pltpu.PARALLEL,),
        )(idx_hbm, out_hbm)

    return gather_kernel(table, flat_idx)


@jax.jit
def kernel(gene_indices, table):
    b, s = gene_indices.shape
    flat_idx = gene_indices.reshape(1, b * s).astype(jnp.int32)
    out = _sc_gather(table, flat_idx)
    return out.reshape(b, s, table.shape[1])


# speedup vs baseline: 1.4989x; 1.0011x over previous
import jax
import jax.numpy as jnp
from jax.experimental import pallas as pl
from jax.experimental.pallas import tpu as pltpu
from jax.experimental.pallas import tpu_sc as plsc

_WINDOW = 1280


def _sc_gather(table, flat_idx):
    num_indices = flat_idx.shape[1]
    d = table.shape[1]
    mesh = plsc.VectorSubcoreMesh(core_axis_name="c", subcore_axis_name="s")

    @pl.kernel(
        out_type=jax.ShapeDtypeStruct((num_indices, d), table.dtype),
        mesh=mesh,
        scratch_types=[pltpu.SemaphoreType.DMA, pltpu.SemaphoreType.DMA],
        compiler_params=pltpu.CompilerParams(use_tc_tiling_on_sc=False),
    )
    def gather_kernel(table_hbm, idx_hbm, out_hbm, sem0, sem1):
        half = _WINDOW // 2

        def body(idx_vmem, out_vmem):
            c0 = pltpu.make_async_copy(
                table_hbm.at[idx_vmem.at[0, pl.ds(0, half)]],
                out_vmem.at[pl.ds(0, half)],
                sem0,
            )
            c1 = pltpu.make_async_copy(
                table_hbm.at[idx_vmem.at[0, pl.ds(half, half)]],
                out_vmem.at[pl.ds(half, half)],
                sem1,
            )
            c0.start()
            c1.start()
            c0.wait()
            c1.wait()

        pltpu.emit_pipeline(
            body,
            grid=(num_indices // _WINDOW,),
            in_specs=[pl.BlockSpec((1, _WINDOW), index_map=lambda i: (0, i))],
            out_specs=[pl.BlockSpec((_WINDOW, d), index_map=lambda i: (i, 0))],
            core_axis_name=("c", "s"),
            dimension_semantics=(pltpu.PARALLEL,),
        )(idx_hbm, out_hbm)

    return gather_kernel(table, flat_idx)


@jax.jit
def kernel(gene_indices, table):
    b, s = gene_indices.shape
    flat_idx = gene_indices.reshape(1, b * s).astype(jnp.int32)
    out = _sc_gather(table, flat_idx)
    return out.reshape(b, s, table.shape[1])


# speedup vs baseline: 1.5020x; 1.0021x over previous
import jax
import jax.numpy as jnp
from jax import lax
from jax.experimental import pallas as pl
from jax.experimental.pallas import tpu as pltpu
from jax.experimental.pallas import tpu_sc as plsc

_CHUNK = 1280
_NTILES = 32


def _sc_gather(table, flat_idx):
    num_indices = flat_idx.shape[1]
    d = table.shape[1]
    per_tile = num_indices // _NTILES
    steps = per_tile // _CHUNK
    mesh = plsc.VectorSubcoreMesh(core_axis_name="c", subcore_axis_name="s")

    @pl.kernel(
        out_type=jax.ShapeDtypeStruct((num_indices, d), table.dtype),
        mesh=mesh,
        scratch_types=[
            pltpu.VMEM((2, _CHUNK), jnp.int32),
            pltpu.VMEM((2, _CHUNK, d), table.dtype),
            pltpu.SemaphoreType.DMA,
            pltpu.SemaphoreType.DMA,
            pltpu.SemaphoreType.DMA,
            pltpu.SemaphoreType.DMA,
            pltpu.SemaphoreType.DMA,
            pltpu.SemaphoreType.DMA,
        ],
        compiler_params=pltpu.CompilerParams(use_tc_tiling_on_sc=False),
    )
    def gather_kernel(
        table_hbm, idx_hbm, out_hbm, idx_v, rows_v, i0, i1, g0, g1, o0, o1
    ):
        isem = (i0, i1)
        gsem = (g0, g1)
        osem = (o0, o1)
        wid = lax.axis_index("s") * 2 + lax.axis_index("c")
        base = wid * per_tile

        def idx_cp(g, b):
            return pltpu.make_async_copy(
                idx_hbm.at[0, pl.ds(base + g * _CHUNK, _CHUNK)],
                idx_v.at[b],
                isem[b],
            )

        def gat_cp(b):
            return pltpu.make_async_copy(
                table_hbm.at[idx_v.at[b]], rows_v.at[b], gsem[b]
            )

        def out_cp(g, b):
            return pltpu.make_async_copy(
                rows_v.at[b],
                out_hbm.at[pl.ds(base + g * _CHUNK, _CHUNK)],
                osem[b],
            )

        idx_cp(0, 0).start()
        idx_cp(1, 1).start()
        idx_cp(0, 0).wait()
        gat_cp(0).start()

        @pl.loop(0, steps, step=2)
        def _(g):
            for b in (0, 1):
                gc = g + b
                ob = 1 - b

                @pl.when(gc + 1 < steps)
                def _():
                    idx_cp(gc + 1, ob).wait()

                    @pl.when(gc >= 1)
                    def _():
                        out_cp(gc - 1, ob).wait()

                    gat_cp(ob).start()

                gat_cp(b).wait()
                out_cp(gc, b).start()

                @pl.when(gc + 2 < steps)
                def _():
                    idx_cp(gc + 2, b).start()

        out_cp(steps - 2, 0).wait()
        out_cp(steps - 1, 1).wait()

    return gather_kernel(table, flat_idx)


@jax.jit
def kernel(gene_indices, table):
    b, s = gene_indices.shape
    flat_idx = gene_indices.reshape(1, b * s).astype(jnp.int32)
    out = _sc_gather(table, flat_idx)
    return out.reshape(b, s, table.shape[1])
